# baseline (speedup 1.0000x reference)
import functools

import jax
import jax.numpy as jnp
from jax import lax
from jax.experimental import pallas as pl
from jax.experimental.pallas import tpu as pltpu
from jax.experimental.pallas import tpu_sc as plsc

B, N = 2, 300
CONTENT_DIM = 256
FEAT_CH = 256
IN_POINTS = 32
OUT_POINTS = 128
N_GROUPS = 4
STRIDES = [4.0, 8.0, 16.0, 32.0]
IMG = 512

_NC, _NS, _LANES = 2, 16, 16
_NW = _NC * _NS
_S_TOT = B * N * N_GROUPS * IN_POINTS
_PER_W = _S_TOT // _NW
_K = 120
_NCHUNK = _PER_W // _K
_NT = 16
_CG = FEAT_CH // N_GROUPS

_SIZES = [IMG // int(s) for s in STRIDES]
_LVL_ROWS = [B * N_GROUPS * hw * hw for hw in _SIZES]

_PERM = []
for _half in range(2):
    for _i in range(16):
        _PERM.append(_half * 32 + _i)
        _PERM.append(_half * 32 + 16 + _i)


def _sc_gather_fn():
    mesh = plsc.VectorSubcoreMesh(
        core_axis_name="c", subcore_axis_name="s",
        num_cores=_NC, num_subcores=_NS)

    @functools.partial(
        pl.kernel,
        out_type=jax.ShapeDtypeStruct((_S_TOT, _CG), jnp.bfloat16),
        mesh=mesh,
        scratch_types=[
            pltpu.VMEM((_NT, _K, _CG), jnp.bfloat16),
            pltpu.VMEM((_K, _CG), jnp.bfloat16),
            pltpu.VMEM((_NT, _K), jnp.int32),
            pltpu.VMEM((_K, _NT), jnp.float32),
            pltpu.SemaphoreType.DMA,
        ],
        compiler_params=pltpu.CompilerParams(use_tc_tiling_on_sc=False,
                                             needs_layout_passes=False),
    )
    def sc_gather(t0, t1, t2, t3, idxs, cws, out, buf, obuf, idxb, cwb, sem):
        tabs = (t0, t1, t2, t3)
        wid = lax.axis_index("s") * _NC + lax.axis_index("c")

        def chunk(j, carry):
            c = wid * _NCHUNK + j
            pltpu.sync_copy(idxs.at[c], idxb)
            pltpu.sync_copy(cws.at[c], cwb)
            handles = [pltpu.async_copy(tabs[t // 4].at[idxb.at[t]],
                                        buf.at[t], sem)
                       for t in range(_NT)]
            for h in handles:
                h.wait()

            def row(k, carry2):
                wrow = cwb[k, :]
                acc = [None] * 4
                for t in range(_NT):
                    w = wrow[t]
                    for h in range(2):
                        v = buf[t, k, pl.ds(h * 32, 32)]
                        a, b2 = plsc.unpack(v, format=plsc.PackFormat.INTERLEAVED)
                        ia, ib = 2 * h, 2 * h + 1
                        acc[ia] = w * a if acc[ia] is None else acc[ia] + w * a
                        acc[ib] = w * b2 if acc[ib] is None else acc[ib] + w * b2
                for h in range(2):
                    packed = plsc.pack(acc[2 * h], acc[2 * h + 1],
                                       format=plsc.PackFormat.INTERLEAVED)
                    obuf[k, pl.ds(h * 32, 32)] = packed
                return carry2

            lax.fori_loop(0, _K, row, 0)
            pltpu.sync_copy(obuf, out.at[pl.ds(wid * _PER_W + j * _K, _K)])
            return carry

        lax.fori_loop(0, _NCHUNK, chunk, 0)

    return sc_gather


_SC_GATHER = _sc_gather_fn()


def _tbuild_body(x_ref, o_ref):
    o_ref[0] = jnp.swapaxes(x_ref[0], 0, 1).astype(jnp.bfloat16)


def _build_tables(xs):
    tabs = []
    for x in xs:
        b, cfull, h, w = x.shape
        hw = h * w
        tile = min(512, hw)
        src = x.reshape(B * N_GROUPS, _CG, hw)
        t = pl.pallas_call(
            _tbuild_body,
            grid=(B * N_GROUPS, hw // tile),
            in_specs=[pl.BlockSpec((1, _CG, tile), lambda i, j: (i, 0, j))],
            out_specs=pl.BlockSpec((1, tile, _CG), lambda i, j: (i, j, 0)),
            out_shape=jax.ShapeDtypeStruct((B * N_GROUPS, hw, _CG),
                                           jnp.bfloat16),
        )(src)
        tabs.append(t.reshape(-1, _CG))
    return tabs


def _build_idx_weights(offset, xyzr):
    off = offset.reshape(B, N, N_GROUPS, IN_POINTS, 3)
    x = xyzr[..., 0][:, :, None, None]
    y = xyzr[..., 1][:, :, None, None]
    z = xyzr[..., 2][:, :, None, None]
    r = xyzr[..., 3][:, :, None, None]
    sx = 2.0 ** (z - 0.5 * r)
    sy = 2.0 ** (z + 0.5 * r)
    px = x + off[..., 0] * sx
    py = y + off[..., 1] * sy
    lvl = z + off[..., 2]

    grid = jnp.log2(jnp.asarray(STRIDES, jnp.float32))
    l2 = -jnp.abs(((lvl[..., None] - grid) ** 2) / 2.0)
    lw = jax.nn.softmax(l2, axis=-1)

    bg = (jnp.arange(B)[:, None, None, None] * N_GROUPS
          + jnp.arange(N_GROUPS)[None, None, :, None])

    idx_list, cw_list = [], []
    for i, stride in enumerate(STRIDES):
        hw = _SIZES[i]
        fx = px / stride - 0.5
        fy = py / stride - 0.5
        x0 = jnp.floor(fx)
        y0 = jnp.floor(fy)
        for dx, dy in ((0, 0), (1, 0), (0, 1), (1, 1)):
            xi = x0 + dx
            yi = y0 + dy
            valid = ((xi >= 0) & (xi <= hw - 1) & (yi >= 0) & (yi <= hw - 1))
            xc = jnp.clip(xi, 0, hw - 1).astype(jnp.int32)
            yc = jnp.clip(yi, 0, hw - 1).astype(jnp.int32)
            gid = (bg * hw + yc) * hw + xc
            wx = (x0 + 1.0 - fx) if dx == 0 else (fx - x0)
            wy = (y0 + 1.0 - fy) if dy == 0 else (fy - y0)
            cw = lw[..., i] * wx * wy * valid.astype(jnp.float32)
            idx_list.append(gid.reshape(-1))
            cw_list.append(cw.reshape(-1))
    idx = jnp.stack(idx_list, axis=0)
    cw = jnp.stack(cw_list, axis=0)
    idx = idx.reshape(_NT, _NW, _NCHUNK, _K).transpose(1, 2, 0, 3).reshape(
        _NW * _NCHUNK, _NT, _K)
    cw = cw.reshape(_NT, _NW, _NCHUNK, _K).transpose(1, 2, 3, 0).reshape(
        _NW * _NCHUNK, _K, _NT)
    return idx, cw


_Q = B * N
_EFF = FEAT_CH // N_GROUPS
_MP = _CG * _EFF
_TOTPG = N_GROUPS * (_MP + IN_POINTS * OUT_POINTS)
_QB = 8
_KT = 2048


_HP = _MP


def _pgemm_body(qf_ref, w_ref, b_ref, pm_ref, ps_ref):
    h = pl.program_id(1)
    r = jnp.dot(qf_ref[...], w_ref[...],
                preferred_element_type=jnp.float32) + b_ref[...]

    @pl.when(h == 0)
    def _wm():
        pm_ref[...] = r[None]

    @pl.when(h == 1)
    def _ws():
        ps_ref[...] = r[None]


def _params_gemm(qf, W_pg, b_pg):
    return pl.pallas_call(
        _pgemm_body,
        grid=(N_GROUPS, 2),
        in_specs=[
            pl.BlockSpec((_Q, CONTENT_DIM), lambda g, h: (0, 0)),
            pl.BlockSpec((CONTENT_DIM, _HP), lambda g, h: (0, g * 2 + h)),
            pl.BlockSpec((1, _HP), lambda g, h: (0, g * 2 + h)),
        ],
        out_specs=[
            pl.BlockSpec((1, _Q, _HP), lambda g, h: (g, 0, 0)),
            pl.BlockSpec((1, _Q, _HP), lambda g, h: (g, 0, 0)),
        ],
        out_shape=[
            jax.ShapeDtypeStruct((N_GROUPS, _Q, _HP), jnp.float32),
            jax.ShapeDtypeStruct((N_GROUPS, _Q, _HP), jnp.float32),
        ],
    )(qf, W_pg, b_pg.reshape(1, _TOTPG))


def _mix_body(x_ref, pm_ref, ps_ref, o_ref, xbd_ref, sbd_ref):
    i = pl.program_id(0)

    @pl.when(i == 0)
    def _zero():
        xbd_ref[...] = jnp.zeros_like(xbd_ref)
        sbd_ref[...] = jnp.zeros_like(sbd_ref)

    for g in range(N_GROUPS):
        for q in range(_QB):
            xbd_ref[pl.ds(q * IN_POINTS, IN_POINTS),
                    pl.ds(q * _CG, _CG)] = x_ref[q, g].astype(jnp.float32)
        mstack = pm_ref[g].reshape(_QB * _CG, _EFF)
        o1 = jnp.dot(xbd_ref[...], mstack,
                     preferred_element_type=jnp.float32)
        o1 = o1.reshape(_QB, IN_POINTS, _EFF)
        mu = jnp.mean(o1, axis=(1, 2), keepdims=True)
        var = jnp.mean((o1 - mu) ** 2, axis=(1, 2), keepdims=True)
        o1 = jax.nn.relu((o1 - mu) * jax.lax.rsqrt(var + 1e-5))
        o1 = o1.reshape(_QB * IN_POINTS, _EFF)
        for q in range(_QB):
            sbd_ref[pl.ds(q * OUT_POINTS, OUT_POINTS),
                    pl.ds(q * IN_POINTS, IN_POINTS)] = ps_ref[g, q]
        o2 = jnp.dot(sbd_ref[...], o1,
                     preferred_element_type=jnp.float32)
        o2 = o2.reshape(_QB, OUT_POINTS, _EFF)
        mu2 = jnp.mean(o2, axis=(1, 2), keepdims=True)
        var2 = jnp.mean((o2 - mu2) ** 2, axis=(1, 2), keepdims=True)
        o2 = jax.nn.relu((o2 - mu2) * jax.lax.rsqrt(var2 + 1e-5))
        for q in range(_QB):
            o_ref[q, g] = o2[q]


def _mixing(sampled, pm, ps):
    grid = (_Q // _QB,)
    o2f = pl.pallas_call(
        _mix_body,
        grid=grid,
        in_specs=[
            pl.BlockSpec((_QB, N_GROUPS, IN_POINTS, _CG), lambda i: (i, 0, 0, 0)),
            pl.BlockSpec((N_GROUPS, _QB, _CG, _EFF), lambda i: (0, i, 0, 0)),
            pl.BlockSpec((N_GROUPS, _QB, OUT_POINTS, IN_POINTS), lambda i: (0, i, 0, 0)),
        ],
        out_specs=pl.BlockSpec((_QB, N_GROUPS, OUT_POINTS, _EFF), lambda i: (i, 0, 0, 0)),
        out_shape=jax.ShapeDtypeStruct((_Q, N_GROUPS, OUT_POINTS, _EFF), jnp.float32),
        scratch_shapes=[
            pltpu.VMEM((_QB * IN_POINTS, _QB * _CG), jnp.float32),
            pltpu.VMEM((_QB * OUT_POINTS, _QB * IN_POINTS), jnp.float32),
        ],
    )(sampled, pm, ps)
    return o2f.reshape(_Q, N_GROUPS * OUT_POINTS * _EFF)


def _out_body(o2_ref, w_ref, qf_ref, b_ref, g_ref, bb_ref, o_ref, acc_ref):
    i = pl.program_id(0)

    @pl.when(i == 0)
    def _init():
        acc_ref[...] = jnp.zeros_like(acc_ref)

    acc_ref[...] += jnp.dot(o2_ref[...], w_ref[...],
                            preferred_element_type=jnp.float32)

    @pl.when(i == pl.num_programs(0) - 1)
    def _fin():
        t = acc_ref[...] + b_ref[...] + qf_ref[...]
        m = jnp.mean(t, axis=-1, keepdims=True)
        v = jnp.mean((t - m) ** 2, axis=-1, keepdims=True)
        o_ref[...] = (t - m) * jax.lax.rsqrt(v + 1e-5) * g_ref[...] + bb_ref[...]


def _out_gemm(o2f, W_out, qf, b_out, ln_g, ln_b):
    grid = (N_GROUPS * OUT_POINTS * _EFF // _KT,)
    return pl.pallas_call(
        _out_body,
        grid=grid,
        in_specs=[
            pl.BlockSpec((_Q, _KT), lambda i: (0, i)),
            pl.BlockSpec((_KT, CONTENT_DIM), lambda i: (i, 0)),
            pl.BlockSpec((_Q, CONTENT_DIM), lambda i: (0, 0)),
            pl.BlockSpec((1, CONTENT_DIM), lambda i: (0, 0)),
            pl.BlockSpec((1, CONTENT_DIM), lambda i: (0, 0)),
            pl.BlockSpec((1, CONTENT_DIM), lambda i: (0, 0)),
        ],
        out_specs=pl.BlockSpec((_Q, CONTENT_DIM), lambda i: (0, 0)),
        out_shape=jax.ShapeDtypeStruct((_Q, CONTENT_DIM), jnp.float32),
        scratch_shapes=[pltpu.VMEM((_Q, CONTENT_DIM), jnp.float32)],
    )(o2f, W_out, qf, b_out.reshape(1, -1), ln_g.reshape(1, -1),
      ln_b.reshape(1, -1))


def kernel(x0, x1, x2, x3, query_feat, query_roi, W_off, b_off, W_pg, b_pg, W_out, b_out, ln_g, ln_b):
    offset = query_feat @ W_off + b_off
    idx, cw = _build_idx_weights(offset, query_roi)
    tabs = _build_tables([x0, x1, x2, x3])
    sampled = _SC_GATHER(*tabs, idx, cw).reshape(_Q, N_GROUPS, IN_POINTS, _CG)
    qf = query_feat.reshape(_Q, CONTENT_DIM)
    pm, ps = _params_gemm(qf, W_pg, b_pg)
    pm = pm.reshape(N_GROUPS, _Q, _CG, _EFF)
    ps = ps.reshape(N_GROUPS, _Q, OUT_POINTS, IN_POINTS)
    o2f = _mixing(sampled, pm, ps)
    out = _out_gemm(o2f, W_out, qf, b_out, ln_g, ln_b)
    return out.reshape(B, N, CONTENT_DIM)


# speedup vs baseline: 1.3333x; 1.3333x over previous
import functools

import jax
import jax.numpy as jnp
from jax import lax
from jax.experimental import pallas as pl
from jax.experimental.pallas import tpu as pltpu
from jax.experimental.pallas import tpu_sc as plsc

B, N = 2, 300
CONTENT_DIM = 256
FEAT_CH = 256
IN_POINTS = 32
OUT_POINTS = 128
N_GROUPS = 4
STRIDES = [4.0, 8.0, 16.0, 32.0]
IMG = 512

_NC, _NS, _LANES = 2, 16, 16
_NW = _NC * _NS
_S_TOT = B * N * N_GROUPS * IN_POINTS
_PER_W = _S_TOT // _NW
_K = 120
_NCHUNK = _PER_W // _K
_NT = 16
_CG = FEAT_CH // N_GROUPS

_SIZES = [IMG // int(s) for s in STRIDES]
_LVL_ROWS = [B * N_GROUPS * hw * hw for hw in _SIZES]

_PERM = []
for _half in range(2):
    for _i in range(16):
        _PERM.append(_half * 32 + _i)
        _PERM.append(_half * 32 + 16 + _i)


def _sc_gather_fn():
    mesh = plsc.VectorSubcoreMesh(
        core_axis_name="c", subcore_axis_name="s",
        num_cores=_NC, num_subcores=_NS)

    @functools.partial(
        pl.kernel,
        out_type=jax.ShapeDtypeStruct((_S_TOT, _CG), jnp.bfloat16),
        mesh=mesh,
        scratch_types=[
            pltpu.VMEM((_NT, _K, _CG), jnp.bfloat16),
            pltpu.VMEM((_K, _CG), jnp.bfloat16),
            pltpu.VMEM((_NT, _K), jnp.int32),
            pltpu.VMEM((_K, _NT), jnp.float32),
            pltpu.SemaphoreType.DMA,
        ],
        compiler_params=pltpu.CompilerParams(use_tc_tiling_on_sc=False,
                                             needs_layout_passes=False),
    )
    def sc_gather(t0, t1, t2, t3, idxs, cws, out, buf, obuf, idxb, cwb, sem):
        tabs = (t0, t1, t2, t3)
        wid = lax.axis_index("s") * _NC + lax.axis_index("c")

        def chunk(j, carry):
            c = wid * _NCHUNK + j
            pltpu.sync_copy(idxs.at[c], idxb)
            pltpu.sync_copy(cws.at[c], cwb)
            handles = [pltpu.async_copy(tabs[t // 4].at[idxb.at[t]],
                                        buf.at[t], sem)
                       for t in range(_NT)]
            for h in handles:
                h.wait()

            def row(k, carry2):
                wrow = cwb[k, :]
                acc = [None] * 4
                for t in range(_NT):
                    w = wrow[t]
                    for h in range(2):
                        v = buf[t, k, pl.ds(h * 32, 32)]
                        a, b2 = plsc.unpack(v, format=plsc.PackFormat.INTERLEAVED)
                        ia, ib = 2 * h, 2 * h + 1
                        acc[ia] = w * a if acc[ia] is None else acc[ia] + w * a
                        acc[ib] = w * b2 if acc[ib] is None else acc[ib] + w * b2
                for h in range(2):
                    packed = plsc.pack(acc[2 * h], acc[2 * h + 1],
                                       format=plsc.PackFormat.INTERLEAVED)
                    obuf[k, pl.ds(h * 32, 32)] = packed
                return carry2

            lax.fori_loop(0, _K, row, 0)
            pltpu.sync_copy(obuf, out.at[pl.ds(wid * _PER_W + j * _K, _K)])
            return carry

        lax.fori_loop(0, _NCHUNK, chunk, 0)

    return sc_gather


_SC_GATHER = _sc_gather_fn()


def _build_tables(xs):
    tabs = []
    for x in xs:
        b, cfull, h, w = x.shape
        t = x.reshape(b, N_GROUPS, _CG, h, w).transpose(0, 1, 3, 4, 2)
        tabs.append(t.reshape(-1, _CG).astype(jnp.bfloat16))
    return tabs


def _build_idx_weights(offset, xyzr):
    off = offset.reshape(B, N, N_GROUPS, IN_POINTS, 3)
    x = xyzr[..., 0][:, :, None, None]
    y = xyzr[..., 1][:, :, None, None]
    z = xyzr[..., 2][:, :, None, None]
    r = xyzr[..., 3][:, :, None, None]
    sx = 2.0 ** (z - 0.5 * r)
    sy = 2.0 ** (z + 0.5 * r)
    px = x + off[..., 0] * sx
    py = y + off[..., 1] * sy
    lvl = z + off[..., 2]

    grid = jnp.log2(jnp.asarray(STRIDES, jnp.float32))
    l2 = -jnp.abs(((lvl[..., None] - grid) ** 2) / 2.0)
    lw = jax.nn.softmax(l2, axis=-1)

    bg = (jnp.arange(B)[:, None, None, None] * N_GROUPS
          + jnp.arange(N_GROUPS)[None, None, :, None])

    idx_list, cw_list = [], []
    for i, stride in enumerate(STRIDES):
        hw = _SIZES[i]
        fx = px / stride - 0.5
        fy = py / stride - 0.5
        x0 = jnp.floor(fx)
        y0 = jnp.floor(fy)
        for dx, dy in ((0, 0), (1, 0), (0, 1), (1, 1)):
            xi = x0 + dx
            yi = y0 + dy
            valid = ((xi >= 0) & (xi <= hw - 1) & (yi >= 0) & (yi <= hw - 1))
            xc = jnp.clip(xi, 0, hw - 1).astype(jnp.int32)
            yc = jnp.clip(yi, 0, hw - 1).astype(jnp.int32)
            gid = (bg * hw + yc) * hw + xc
            wx = (x0 + 1.0 - fx) if dx == 0 else (fx - x0)
            wy = (y0 + 1.0 - fy) if dy == 0 else (fy - y0)
            cw = lw[..., i] * wx * wy * valid.astype(jnp.float32)
            idx_list.append(gid.reshape(-1))
            cw_list.append(cw.reshape(-1))
    idx = jnp.stack(idx_list, axis=0)
    cw = jnp.stack(cw_list, axis=0)
    idx = idx.reshape(_NT, _NW, _NCHUNK, _K).transpose(1, 2, 0, 3).reshape(
        _NW * _NCHUNK, _NT, _K)
    cw = cw.reshape(_NT, _NW, _NCHUNK, _K).transpose(1, 2, 3, 0).reshape(
        _NW * _NCHUNK, _K, _NT)
    return idx, cw


_Q = B * N
_EFF = FEAT_CH // N_GROUPS
_MP = _CG * _EFF
_TOTPG = N_GROUPS * (_MP + IN_POINTS * OUT_POINTS)
_QB = 8
_KT = 2048


_HP = _MP


def _pgemm_body(qf_ref, w_ref, b_ref, pm_ref, ps_ref):
    h = pl.program_id(1)
    r = jnp.dot(qf_ref[...], w_ref[...],
                preferred_element_type=jnp.float32) + b_ref[...]

    @pl.when(h == 0)
    def _wm():
        pm_ref[...] = r[None].astype(jnp.bfloat16)

    @pl.when(h == 1)
    def _ws():
        ps_ref[...] = r[None].astype(jnp.bfloat16)


def _params_gemm(qf, W_pg, b_pg):
    return pl.pallas_call(
        _pgemm_body,
        grid=(N_GROUPS, 2),
        in_specs=[
            pl.BlockSpec((_Q, CONTENT_DIM), lambda g, h: (0, 0)),
            pl.BlockSpec((CONTENT_DIM, _HP), lambda g, h: (0, g * 2 + h)),
            pl.BlockSpec((1, _HP), lambda g, h: (0, g * 2 + h)),
        ],
        out_specs=[
            pl.BlockSpec((1, _Q, _HP), lambda g, h: (g, 0, 0)),
            pl.BlockSpec((1, _Q, _HP), lambda g, h: (g, 0, 0)),
        ],
        out_shape=[
            jax.ShapeDtypeStruct((N_GROUPS, _Q, _HP), jnp.bfloat16),
            jax.ShapeDtypeStruct((N_GROUPS, _Q, _HP), jnp.bfloat16),
        ],
    )(qf, W_pg, b_pg.reshape(1, _TOTPG))


def _mix_body(x_ref, pm_ref, ps_ref, o_ref, xbd_ref, sbd_ref):
    i = pl.program_id(0)

    @pl.when(i == 0)
    def _zero():
        xbd_ref[...] = jnp.zeros_like(xbd_ref)
        sbd_ref[...] = jnp.zeros_like(sbd_ref)

    for g in range(N_GROUPS):
        for q in range(_QB):
            xbd_ref[pl.ds(q * IN_POINTS, IN_POINTS),
                    pl.ds(q * _CG, _CG)] = x_ref[q, g]
        mstack = pm_ref[g].reshape(_QB * _CG, _EFF)
        o1 = jnp.dot(xbd_ref[...], mstack,
                     preferred_element_type=jnp.float32)
        o1 = o1.reshape(_QB, IN_POINTS, _EFF)
        mu = jnp.mean(o1, axis=(1, 2), keepdims=True)
        var = jnp.mean((o1 - mu) ** 2, axis=(1, 2), keepdims=True)
        o1 = jax.nn.relu((o1 - mu) * jax.lax.rsqrt(var + 1e-5))
        o1 = o1.reshape(_QB * IN_POINTS, _EFF).astype(jnp.bfloat16)
        for q in range(_QB):
            sbd_ref[pl.ds(q * OUT_POINTS, OUT_POINTS),
                    pl.ds(q * IN_POINTS, IN_POINTS)] = ps_ref[g, q]
        o2 = jnp.dot(sbd_ref[...], o1,
                     preferred_element_type=jnp.float32)
        o2 = o2.reshape(_QB, OUT_POINTS, _EFF)
        mu2 = jnp.mean(o2, axis=(1, 2), keepdims=True)
        var2 = jnp.mean((o2 - mu2) ** 2, axis=(1, 2), keepdims=True)
        o2 = jax.nn.relu((o2 - mu2) * jax.lax.rsqrt(var2 + 1e-5))
        o2 = o2.astype(jnp.bfloat16)
        for q in range(_QB):
            o_ref[q, g] = o2[q]


def _mixing(sampled, pm, ps):
    grid = (_Q // _QB,)
    o2f = pl.pallas_call(
        _mix_body,
        grid=grid,
        in_specs=[
            pl.BlockSpec((_QB, N_GROUPS, IN_POINTS, _CG), lambda i: (i, 0, 0, 0)),
            pl.BlockSpec((N_GROUPS, _QB, _CG, _EFF), lambda i: (0, i, 0, 0)),
            pl.BlockSpec((N_GROUPS, _QB, OUT_POINTS, IN_POINTS), lambda i: (0, i, 0, 0)),
        ],
        out_specs=pl.BlockSpec((_QB, N_GROUPS, OUT_POINTS, _EFF), lambda i: (i, 0, 0, 0)),
        out_shape=jax.ShapeDtypeStruct((_Q, N_GROUPS, OUT_POINTS, _EFF), jnp.bfloat16),
        scratch_shapes=[
            pltpu.VMEM((_QB * IN_POINTS, _QB * _CG), jnp.bfloat16),
            pltpu.VMEM((_QB * OUT_POINTS, _QB * IN_POINTS), jnp.bfloat16),
        ],
    )(sampled, pm, ps)
    return o2f.reshape(_Q, N_GROUPS * OUT_POINTS * _EFF)


def _out_body(o2_ref, w_ref, qf_ref, b_ref, g_ref, bb_ref, o_ref, acc_ref):
    i = pl.program_id(0)

    @pl.when(i == 0)
    def _init():
        acc_ref[...] = jnp.zeros_like(acc_ref)

    acc_ref[...] += jnp.dot(o2_ref[...], w_ref[...],
                            preferred_element_type=jnp.float32)

    @pl.when(i == pl.num_programs(0) - 1)
    def _fin():
        t = acc_ref[...] + b_ref[...] + qf_ref[...]
        m = jnp.mean(t, axis=-1, keepdims=True)
        v = jnp.mean((t - m) ** 2, axis=-1, keepdims=True)
        o_ref[...] = (t - m) * jax.lax.rsqrt(v + 1e-5) * g_ref[...] + bb_ref[...]


def _out_gemm(o2f, W_out, qf, b_out, ln_g, ln_b):
    grid = (N_GROUPS * OUT_POINTS * _EFF // _KT,)
    return pl.pallas_call(
        _out_body,
        grid=grid,
        in_specs=[
            pl.BlockSpec((_Q, _KT), lambda i: (0, i)),
            pl.BlockSpec((_KT, CONTENT_DIM), lambda i: (i, 0)),
            pl.BlockSpec((_Q, CONTENT_DIM), lambda i: (0, 0)),
            pl.BlockSpec((1, CONTENT_DIM), lambda i: (0, 0)),
            pl.BlockSpec((1, CONTENT_DIM), lambda i: (0, 0)),
            pl.BlockSpec((1, CONTENT_DIM), lambda i: (0, 0)),
        ],
        out_specs=pl.BlockSpec((_Q, CONTENT_DIM), lambda i: (0, 0)),
        out_shape=jax.ShapeDtypeStruct((_Q, CONTENT_DIM), jnp.float32),
        scratch_shapes=[pltpu.VMEM((_Q, CONTENT_DIM), jnp.float32)],
    )(o2f, W_out.astype(jnp.bfloat16), qf, b_out.reshape(1, -1),
      ln_g.reshape(1, -1), ln_b.reshape(1, -1))


def kernel(x0, x1, x2, x3, query_feat, query_roi, W_off, b_off, W_pg, b_pg, W_out, b_out, ln_g, ln_b):
    offset = query_feat @ W_off + b_off
    idx, cw = _build_idx_weights(offset, query_roi)
    tabs = _build_tables([x0, x1, x2, x3])
    sampled = _SC_GATHER(*tabs, idx, cw).reshape(_Q, N_GROUPS, IN_POINTS, _CG)
    qf = query_feat.reshape(_Q, CONTENT_DIM)
    pm, ps = _params_gemm(qf, W_pg, b_pg)
    pm = pm.reshape(N_GROUPS, _Q, _CG, _EFF)
    ps = ps.reshape(N_GROUPS, _Q, OUT_POINTS, IN_POINTS)
    o2f = _mixing(sampled, pm, ps)
    out = _out_gemm(o2f, W_out, qf, b_out, ln_g, ln_b)
    return out.reshape(B, N, CONTENT_DIM)
